# Initial kernel scaffold; baseline (speedup 1.0000x reference)
import jax
import jax.numpy as jnp
from jax.experimental import pallas as pl


def kernel(features, edge_index, edge_weights, W1, W2, Wc, gamma1, beta1, alpha1, gamma2, beta2, alpha2):
    raise NotImplementedError("write your pallas kernel here")


# speedup vs baseline: 1.0492x; 1.0492x over previous
import functools

import jax
import jax.numpy as jnp
from jax.experimental import pallas as pl
from jax.experimental.pallas import tpu as pltpu

N = 10000
E = 320000
D = 128
OUT = 10
EPS = 1e-5
SLOPE = 0.01


def _prep_body(deg_out_ref, deg_in_ref, x_ref, h0_ref, ndst_ref):
    deg_out = deg_out_ref[...]
    deg_in = deg_in_ref[...]
    nsrc = jax.lax.rsqrt(jnp.maximum(deg_out, 1.0))
    ndst_ref[...] = jax.lax.rsqrt(jnp.maximum(deg_in, 1.0))
    h0_ref[...] = x_ref[...] * nsrc


def _prep(deg_out, deg_in, x):
    return pl.pallas_call(
        _prep_body,
        out_shape=(
            jax.ShapeDtypeStruct((N, D), jnp.float32),
            jax.ShapeDtypeStruct((N, 1), jnp.float32),
        ),
    )(deg_out, deg_in, x)


def _layer_body2(agg_ref, ndst_ref, nsrc_ref, w_ref, gamma_ref, beta_ref,
                 alpha_ref, hs_ref, r_ref):
    agg = agg_ref[...] * ndst_ref[...]
    y = jnp.dot(agg, w_ref[...], preferred_element_type=jnp.float32)
    mean = jnp.mean(y, axis=0, keepdims=True)
    xc = y - alpha_ref[...] * mean
    var = jnp.mean(xc * xc, axis=0, keepdims=True)
    h = gamma_ref[...] * xc * jax.lax.rsqrt(var + EPS) + beta_ref[...]
    h = jnp.where(h >= 0.0, h, SLOPE * h)
    r_ref[...] = jnp.mean(h, axis=0, keepdims=True)
    hs_ref[...] = h * nsrc_ref[...]


def _layer1(agg, ndst, nsrc, w, gamma, beta, alpha):
    return pl.pallas_call(
        _layer_body2,
        out_shape=(
            jax.ShapeDtypeStruct((N, D), jnp.float32),
            jax.ShapeDtypeStruct((1, D), jnp.float32),
        ),
    )(agg, ndst, nsrc, w, gamma, beta, alpha)


def _final_body(agg_ref, ndst_ref, w_ref, gamma_ref, beta_ref, alpha_ref,
                r1_ref, wc_ref, out_ref):
    agg = agg_ref[...] * ndst_ref[...]
    y = jnp.dot(agg, w_ref[...], preferred_element_type=jnp.float32)
    mean = jnp.mean(y, axis=0, keepdims=True)
    xc = y - alpha_ref[...] * mean
    var = jnp.mean(xc * xc, axis=0, keepdims=True)
    h = gamma_ref[...] * xc * jax.lax.rsqrt(var + EPS) + beta_ref[...]
    h = jnp.where(h >= 0.0, h, SLOPE * h)
    r2 = jnp.mean(h, axis=0, keepdims=True)
    r = jnp.concatenate([r1_ref[...], r2], axis=1)
    out_ref[...] = jnp.dot(r, wc_ref[...], preferred_element_type=jnp.float32)


def _final(agg, ndst, w, gamma, beta, alpha, r1, wc):
    return pl.pallas_call(
        _final_body,
        out_shape=jax.ShapeDtypeStruct((1, OUT), jnp.float32),
    )(agg, ndst, w, gamma, beta, alpha, r1, wc)


def kernel(features, edge_index, edge_weights, W1, W2, Wc,
           gamma1, beta1, alpha1, gamma2, beta2, alpha2):
    src = edge_index[0]
    dst = edge_index[1]

    deg_out = jnp.zeros((N,), jnp.float32).at[src].add(1.0)
    deg_in = jnp.zeros((N,), jnp.float32).at[dst].add(1.0)

    h0, ndst = _prep(deg_out[:, None], deg_in[:, None], features)

    msg = h0[src] * edge_weights[:, None]
    agg1 = jax.ops.segment_sum(msg, dst, num_segments=N)

    nsrc = jax.lax.rsqrt(jnp.maximum(deg_out, 1.0))[:, None]
    h1s, r1 = _layer1(agg1, ndst, nsrc, W1, gamma1[None, :], beta1[None, :],
                      alpha1[None, :])

    msg2 = h1s[src] * edge_weights[:, None]
    agg2 = jax.ops.segment_sum(msg2, dst, num_segments=N)

    return _final(agg2, ndst, W2, gamma2[None, :], beta2[None, :],
                  alpha2[None, :], r1, Wc)


# speedup vs baseline: 4.5860x; 4.3711x over previous
import jax
import jax.numpy as jnp
from jax import lax
from jax.experimental import pallas as pl
from jax.experimental.pallas import tpu as pltpu
from jax.experimental.pallas import tpu_sc as plsc

N = 10000
NP = 10240
E = 320000
D = 128
OUT = 10
EPS = 1e-5
SLOPE = 0.01

NC = 2
NS = 16
CH = 128
CPW = 80
EPW = CH * CPW
EP = EPW * NC * NS
RSTRIPE = NP // NS

_mesh = plsc.VectorSubcoreMesh(core_axis_name="c", subcore_axis_name="s",
                               num_cores=NC, num_subcores=NS)


def _deg_body(src_ref, dst_ref, val_ref, out_ref, idx_v, val_v, z_v,
              degs_sh, degd_sh):
    t = lax.axis_index("s")
    cc = lax.axis_index("c")
    wid = t * NC + cc

    @pl.loop(0, RSTRIPE // 16)
    def _zero(i):
        z_v[pl.ds(i * 16, 16)] = jnp.zeros((16,), jnp.float32)

    pltpu.sync_copy(z_v, degs_sh.at[pl.ds(t * RSTRIPE, RSTRIPE)])
    pltpu.sync_copy(z_v, degd_sh.at[pl.ds(t * RSTRIPE, RSTRIPE)])
    plsc.subcore_barrier()

    base = wid * EPW

    @pl.loop(0, CPW)
    def _chunk(ci):
        off = base + ci * CH
        pltpu.sync_copy(val_ref.at[pl.ds(off, CH)], val_v)
        pltpu.sync_copy(src_ref.at[pl.ds(off, CH)], idx_v)
        pltpu.sync_copy(val_v, degs_sh.at[idx_v], add=True)
        pltpu.sync_copy(dst_ref.at[pl.ds(off, CH)], idx_v)
        pltpu.sync_copy(val_v, degd_sh.at[idx_v], add=True)

    plsc.subcore_barrier()
    row = cc * 2 * NP + t * RSTRIPE
    pltpu.sync_copy(degs_sh.at[pl.ds(t * RSTRIPE, RSTRIPE)],
                    out_ref.at[pl.ds(row, RSTRIPE)])
    pltpu.sync_copy(degd_sh.at[pl.ds(t * RSTRIPE, RSTRIPE)],
                    out_ref.at[pl.ds(row + NP, RSTRIPE)])


def _degrees(src_p, dst_p, ones_p):
    k = pl.kernel(
        _deg_body,
        out_type=jax.ShapeDtypeStruct((4 * NP,), jnp.float32),
        mesh=_mesh,
        scratch_types=[
            pltpu.VMEM((CH,), jnp.int32),
            pltpu.VMEM((CH,), jnp.float32),
            pltpu.VMEM((RSTRIPE,), jnp.float32),
            pltpu.VMEM_SHARED((NP,), jnp.float32),
            pltpu.VMEM_SHARED((NP,), jnp.float32),
        ],
    )
    return k(src_p, dst_p, ones_p)


def _edge_body(h_ref, src_ref, dst_ref, ew_ref, out_ref,
               srcv, dstv, ewv, rows, acc_sh, gsem):
    t = lax.axis_index("s")
    cc = lax.axis_index("c")
    wid = t * NC + cc

    @pl.loop(0, CH)
    def _zrow(i):
        for f in range(D // 16):
            rows[i, pl.ds(f * 16, 16)] = jnp.zeros((16,), jnp.float32)

    for i in range(RSTRIPE // CH):
        pltpu.sync_copy(rows, acc_sh.at[pl.ds(t * RSTRIPE + i * CH, CH)])
    plsc.subcore_barrier()

    base = wid * EPW

    @pl.loop(0, CPW)
    def _chunk(ci):
        off = base + ci * CH
        pltpu.sync_copy(src_ref.at[pl.ds(off, CH)], srcv)
        pltpu.sync_copy(dst_ref.at[pl.ds(off, CH)], dstv)
        pltpu.sync_copy(ew_ref.at[pl.ds(off, CH)], ewv)
        pltpu.async_copy(h_ref.at[srcv], rows, gsem).wait()
        for g in range(CH // 16):
            ew16 = ewv[pl.ds(g * 16, 16)]
            for j in range(16):
                e = g * 16 + j
                ewb = lax.gather(
                    ew16, jnp.full((16, 1), j, jnp.int32),
                    lax.GatherDimensionNumbers(
                        offset_dims=(), collapsed_slice_dims=(0,),
                        start_index_map=(0,)),
                    (1,), mode=lax.GatherScatterMode.PROMISE_IN_BOUNDS)
                for f in range(D // 16):
                    sl = pl.ds(f * 16, 16)
                    rows[e, sl] = rows[e, sl] * ewb
        pltpu.sync_copy(rows, acc_sh.at[dstv], add=True)

    plsc.subcore_barrier()
    pltpu.sync_copy(acc_sh.at[pl.ds(t * RSTRIPE, RSTRIPE)],
                    out_ref.at[pl.ds(cc * NP + t * RSTRIPE, RSTRIPE)])


def _edge_pass(h, src_p, dst_p, ew_p):
    k = pl.kernel(
        _edge_body,
        out_type=jax.ShapeDtypeStruct((2 * NP, D), jnp.float32),
        mesh=_mesh,
        scratch_types=[
            pltpu.VMEM((CH,), jnp.int32),
            pltpu.VMEM((CH,), jnp.int32),
            pltpu.VMEM((CH,), jnp.float32),
            pltpu.VMEM((CH, D), jnp.float32),
            pltpu.VMEM_SHARED((NP, D), jnp.float32),
            pltpu.SemaphoreType.DMA,
        ],
    )
    return k(h, src_p, dst_p, ew_p)


def _prep_body(deg_ref, x_ref, h0_ref, nsrc_ref, ndst_ref):
    deg = deg_ref[...]
    dsrc = deg[:, 0:1] + deg[:, 2:3]
    ddst = deg[:, 1:2] + deg[:, 3:4]
    nsrc = lax.rsqrt(jnp.maximum(dsrc, 1.0))
    nsrc_ref[...] = nsrc
    ndst_ref[...] = lax.rsqrt(jnp.maximum(ddst, 1.0))
    h0_ref[...] = x_ref[...] * nsrc


def _prep(deg4, x_pad):
    return pl.pallas_call(
        _prep_body,
        out_shape=(
            jax.ShapeDtypeStruct((NP, D), jnp.float32),
            jax.ShapeDtypeStruct((NP, 1), jnp.float32),
            jax.ShapeDtypeStruct((NP, 1), jnp.float32),
        ),
    )(deg4, x_pad)


def _dense_layer(p_ref, ndst_ref, w_ref, g_ref, b_ref, a_ref):
    p = p_ref[...]
    agg = (p[:NP] + p[NP:]) * ndst_ref[...]
    y = jnp.dot(agg, w_ref[...], preferred_element_type=jnp.float32)
    mask = lax.broadcasted_iota(jnp.int32, (NP, 1), 0) < N
    mean = jnp.sum(y, axis=0, keepdims=True) * (1.0 / N)
    xc = y - a_ref[...] * mean
    xcm = jnp.where(mask, xc, 0.0)
    var = jnp.sum(xcm * xcm, axis=0, keepdims=True) * (1.0 / N)
    h = g_ref[...] * xc * lax.rsqrt(var + EPS) + b_ref[...]
    h = jnp.where(h >= 0.0, h, SLOPE * h)
    return jnp.where(mask, h, 0.0)


def _layer_body(p_ref, ndst_ref, nsrc_ref, w_ref, g_ref, b_ref, a_ref,
                hs_ref, r_ref):
    h = _dense_layer(p_ref, ndst_ref, w_ref, g_ref, b_ref, a_ref)
    r_ref[...] = jnp.sum(h, axis=0, keepdims=True) * (1.0 / N)
    hs_ref[...] = h * nsrc_ref[...]


def _layer(partials, ndst, nsrc, w, gamma, beta, alpha):
    return pl.pallas_call(
        _layer_body,
        out_shape=(
            jax.ShapeDtypeStruct((NP, D), jnp.float32),
            jax.ShapeDtypeStruct((1, D), jnp.float32),
        ),
    )(partials, ndst, nsrc, w, gamma, beta, alpha)


def _final_body(p_ref, ndst_ref, w_ref, g_ref, b_ref, a_ref, r1_ref, wc_ref,
                out_ref):
    h = _dense_layer(p_ref, ndst_ref, w_ref, g_ref, b_ref, a_ref)
    r2 = jnp.sum(h, axis=0, keepdims=True) * (1.0 / N)
    r = jnp.concatenate([r1_ref[...], r2], axis=1)
    out_ref[...] = jnp.dot(r, wc_ref[...], preferred_element_type=jnp.float32)


def _final(partials, ndst, w, gamma, beta, alpha, r1, wc):
    return pl.pallas_call(
        _final_body,
        out_shape=jax.ShapeDtypeStruct((1, OUT), jnp.float32),
    )(partials, ndst, w, gamma, beta, alpha, r1, wc)


def kernel(features, edge_index, edge_weights, W1, W2, Wc,
           gamma1, beta1, alpha1, gamma2, beta2, alpha2):
    src = edge_index[0]
    dst = edge_index[1]

    pad = EP - E
    pad_idx = (N + (jnp.arange(pad, dtype=jnp.int32) % (NP - N)))
    src_p = jnp.concatenate([src, pad_idx])
    dst_p = jnp.concatenate([dst, pad_idx])
    ew_p = jnp.concatenate([edge_weights, jnp.zeros((pad,), jnp.float32)])
    ones_p = jnp.concatenate(
        [jnp.ones((E,), jnp.float32), jnp.zeros((pad,), jnp.float32)])
    x_pad = jnp.pad(features, ((0, NP - N), (0, 0)))

    degflat = _degrees(src_p, dst_p, ones_p)
    deg4 = degflat.reshape(4, NP).T

    h0, nsrc, ndst = _prep(deg4, x_pad)
    p1 = _edge_pass(h0, src_p, dst_p, ew_p)
    h1s, r1 = _layer(p1, ndst, nsrc, W1, gamma1[None, :], beta1[None, :],
                     alpha1[None, :])
    p2 = _edge_pass(h1s, src_p, dst_p, ew_p)
    return _final(p2, ndst, W2, gamma2[None, :], beta2[None, :],
                  alpha2[None, :], r1, Wc)
